# baseline (speedup 1.0000x reference)
import functools

import jax
import jax.numpy as jnp
from jax import lax
from jax.experimental import pallas as pl
from jax.experimental.pallas import tpu as pltpu, tpu_sc as plsc

BATCH = 16384
EMB_DIM = 64
NUM_REL = 1001
NUM_WORKERS = 32
B_PER_W = BATCH // NUM_WORKERS
CHUNK = 128
N_CHUNKS = B_PER_W // CHUNK


def _transe_gather(h, r, t, ent_emb, rel_emb):
    mesh = plsc.VectorSubcoreMesh(core_axis_name="c", subcore_axis_name="s")

    out_t = jax.ShapeDtypeStruct((EMB_DIM, BATCH), jnp.float32)
    buf_t = lambda: pltpu.VMEM((EMB_DIM, CHUNK), jnp.float32)

    @functools.partial(
        pl.kernel,
        mesh=mesh,
        compiler_params=pltpu.CompilerParams(use_tc_tiling_on_sc=False,
                                             needs_layout_passes=False,
                                             disable_bounds_checks=True),
        out_type=[out_t, out_t, out_t],
        scratch_types=[
            pltpu.VMEM((NUM_REL, EMB_DIM), jnp.float32),
            [pltpu.VMEM((CHUNK, EMB_DIM), jnp.float32),
             pltpu.VMEM((CHUNK, EMB_DIM), jnp.float32)],
            [buf_t(), buf_t(), buf_t()],
            [pltpu.VMEM((B_PER_W,), jnp.int32) for _ in range(3)],
            pltpu.SemaphoreType.DMA,
            pltpu.SemaphoreType.DMA,
        ],
    )
    def k(h_hbm, r_hbm, t_hbm, ent_hbm, rel_hbm,
          h_out, r_out, t_out,
          relv, rowb, bufs, vidx, gsem, wsem):
        wid = lax.axis_index("s") * 2 + lax.axis_index("c")
        base = wid * B_PER_W
        lane = lax.iota(jnp.int32, 16)

        for j, src in enumerate((h_hbm, r_hbm, t_hbm)):
            pltpu.sync_copy(src.at[pl.ds(base, B_PER_W)], vidx[j])
        pltpu.sync_copy(rel_hbm, relv)

        for c in range(N_CHUNKS):
            off = base + c * CHUNK

            if c > 0:
                for j in range(3):
                    pltpu.make_async_copy(
                        bufs[j], h_out.at[:, pl.ds(0, CHUNK)], wsem).wait()

            def fire(g, _):
                eh = vidx[0][pl.ds(c * CHUNK + g * 16, 16)]
                et = vidx[2][pl.ds(c * CHUNK + g * 16, 16)]
                for ln in range(16):
                    pltpu.async_copy(ent_hbm.at[eh[ln]],
                                     rowb[0].at[g * 16 + ln], gsem)
                    pltpu.async_copy(ent_hbm.at[et[ln]],
                                     rowb[1].at[g * 16 + ln], gsem)
                return 0
            lax.fori_loop(0, CHUNK // 16, fire, 0)

            def rgather(g, _):
                er = vidx[1][pl.ds(c * CHUNK + g * 16, 16)]
                for f in range(EMB_DIM):
                    fv = jnp.full((16,), f, jnp.int32)
                    bufs[1][f, pl.ds(g * 16, 16)] = plsc.load_gather(
                        relv, [er, fv])
                return 0
            lax.fori_loop(0, CHUNK // 16, rgather, 0)
            pltpu.async_copy(bufs[1], r_out.at[:, pl.ds(off, CHUNK)], wsem)

            def drain(i, _):
                pltpu.make_async_copy(ent_hbm.at[0], rowb[0].at[0],
                                      gsem).wait()
                pltpu.make_async_copy(ent_hbm.at[0], rowb[1].at[0],
                                      gsem).wait()
                return 0
            lax.fori_loop(0, CHUNK, drain, 0)

            for jj, (jb, out) in enumerate(((0, h_out), (2, t_out))):
                def flip(g, _, jj=jj, jb=jb):
                    iv = lane + g * 16
                    for f in range(EMB_DIM):
                        fv = jnp.full((16,), f, jnp.int32)
                        bufs[jb][f, pl.ds(g * 16, 16)] = plsc.load_gather(
                            rowb[jj], [iv, fv])
                    return 0
                lax.fori_loop(0, CHUNK // 16, flip, 0)
                pltpu.async_copy(bufs[jb], out.at[:, pl.ds(off, CHUNK)], wsem)

        for j in range(3):
            pltpu.make_async_copy(
                bufs[j], h_out.at[:, pl.ds(0, CHUNK)], wsem).wait()

    return k(h, r, t, ent_emb, rel_emb)


def kernel(h, r, t, ent_emb, rel_emb):
    h = h.astype(jnp.int32)
    r = r.astype(jnp.int32)
    t = t.astype(jnp.int32)
    h_t, r_t, t_t = _transe_gather(h, r, t, ent_emb, rel_emb)
    return (h_t.T, r_t.T, t_t.T)


# speedup vs baseline: 1.8228x; 1.8228x over previous
import functools

import jax
import jax.numpy as jnp
from jax import lax
from jax.experimental import pallas as pl
from jax.experimental.pallas import tpu as pltpu, tpu_sc as plsc

BATCH = 16384
EMB_DIM = 64
NUM_WORKERS = 32
B_PER_W = BATCH // NUM_WORKERS
CHUNK = 128
N_CHUNKS = B_PER_W // CHUNK


def _transe_gather(h, r, t, ent_emb, rel_emb):
    mesh = plsc.VectorSubcoreMesh(core_axis_name="c", subcore_axis_name="s")

    row_buf = lambda: pltpu.VMEM((CHUNK, EMB_DIM), jnp.float32)

    @functools.partial(
        pl.kernel,
        mesh=mesh,
        compiler_params=pltpu.CompilerParams(use_tc_tiling_on_sc=True),
        out_type=[
            jax.ShapeDtypeStruct((BATCH, EMB_DIM), jnp.float32),
            jax.ShapeDtypeStruct((BATCH, EMB_DIM), jnp.float32),
            jax.ShapeDtypeStruct((BATCH, EMB_DIM), jnp.float32),
        ],
        scratch_types=[
            [[row_buf(), row_buf(), row_buf()] for _ in range(2)],
            [pltpu.VMEM((B_PER_W,), jnp.int32) for _ in range(3)],
            pltpu.SemaphoreType.DMA,
            pltpu.SemaphoreType.DMA,
        ],
    )
    def k(h_hbm, r_hbm, t_hbm, ent_hbm, rel_hbm,
          h_out, r_out, t_out,
          rows, vidx, gsem, wsem):
        wid = lax.axis_index("s") * 2 + lax.axis_index("c")
        base = wid * B_PER_W
        idx_srcs = (h_hbm, r_hbm, t_hbm)
        tabs = (ent_hbm, rel_hbm, ent_hbm)
        outs = (h_out, r_out, t_out)

        for j in range(3):
            pltpu.sync_copy(idx_srcs[j].at[pl.ds(base, B_PER_W)], vidx[j])

        for c in range(N_CHUNKS):
            b = c % 2
            csl = pl.ds(base + c * CHUNK, CHUNK)

            if c >= 2:
                for j in range(3):
                    pltpu.make_async_copy(rows[b][j], outs[j].at[csl],
                                          wsem).wait()

            def fire(g, _, b=b):
                for j in range(3):
                    vec = vidx[j][pl.ds(c * CHUNK + g * 16, 16)]
                    for lane in range(16):
                        pltpu.async_copy(tabs[j].at[vec[lane]],
                                         rows[b][j].at[g * 16 + lane], gsem)
                return 0
            lax.fori_loop(0, CHUNK // 16, fire, 0)

            def drain(i, _, b=b):
                for j in range(3):
                    pltpu.make_async_copy(tabs[j].at[0], rows[b][j].at[0],
                                          gsem).wait()
                return 0
            lax.fori_loop(0, CHUNK, drain, 0)

            for j in range(3):
                pltpu.async_copy(rows[b][j], outs[j].at[csl], wsem)

        for c in range(max(0, N_CHUNKS - 2), N_CHUNKS):
            b = c % 2
            csl = pl.ds(base + c * CHUNK, CHUNK)
            for j in range(3):
                pltpu.make_async_copy(rows[b][j], outs[j].at[csl],
                                      wsem).wait()

    return k(h, r, t, ent_emb, rel_emb)


def kernel(h, r, t, ent_emb, rel_emb):
    h = h.astype(jnp.int32)
    r = r.astype(jnp.int32)
    t = t.astype(jnp.int32)
    h_e, r_e, t_e = _transe_gather(h, r, t, ent_emb, rel_emb)
    return (h_e, r_e, t_e)
